# baseline (speedup 1.0000x reference)
import jax
import jax.numpy as jnp
from jax import lax
from jax.experimental import pallas as pl
from jax.experimental.pallas import tpu as pltpu

_N = 10000
_NPAD = 10240


def _body(x_ref, o_ref):
    x = x_ref[...]
    r = lax.broadcasted_iota(jnp.int32, (5, _NPAD), 0)
    c = lax.broadcasted_iota(jnp.int32, (5, _NPAD), 1)
    conf = jnp.where(r == 4, x, 1.0)
    badm = jnp.logical_and(c < _N, conf < 0.0)
    cbad = jnp.min(jnp.where(badm, c, _N))
    tot = jnp.sum(jnp.where(c < cbad, x, 0.0))
    o_ref[...] = jnp.full((1, 1), tot, jnp.float32)


def kernel(data):
    big = jnp.transpose(data[0, :_NPAD, 0:5])
    out = pl.pallas_call(
        _body,
        grid=(1,),
        in_specs=[pl.BlockSpec(memory_space=pltpu.MemorySpace.VMEM)],
        out_specs=pl.BlockSpec((1, 1), lambda i: (0, 0)),
        out_shape=jax.ShapeDtypeStruct((1, 1), jnp.float32),
    )(big)
    return out[0, 0]


# speedup vs baseline: 1.0528x; 1.0528x over previous
import jax
import jax.numpy as jnp
from jax import lax
from jax.experimental import pallas as pl
from jax.experimental.pallas import tpu as pltpu

_N = 10000
_NPAD = 10240


def _body(x_ref, o_ref):
    pos = (lax.broadcasted_iota(jnp.int32, (80, 128), 0) * 128
           + lax.broadcasted_iota(jnp.int32, (80, 128), 1))
    x4 = x_ref[4]
    badm = jnp.logical_and(pos < _N, x4 < 0.0)
    cbad = jnp.min(jnp.where(badm, pos, _N))
    s = x_ref[0] + x_ref[1] + x_ref[2] + x_ref[3] + x4
    tot = jnp.sum(jnp.where(pos < cbad, s, 0.0))
    o_ref[...] = jnp.full((1, 1), tot, jnp.float32)


def kernel(data):
    big = jnp.transpose(data[0, :_NPAD, 0:5])
    out = pl.pallas_call(
        _body,
        grid=(1,),
        in_specs=[pl.BlockSpec(memory_space=pltpu.MemorySpace.VMEM)],
        out_specs=pl.BlockSpec((1, 1), lambda i: (0, 0)),
        out_shape=jax.ShapeDtypeStruct((1, 1), jnp.float32),
    )(big.reshape(5, 80, 128))
    return out[0, 0]
